# baseline (speedup 1.0000x reference)
import functools

import jax
import jax.numpy as jnp
from jax import lax
from jax.experimental import pallas as pl
from jax.experimental.pallas import tpu as pltpu


def _nearest_idx_kernel(x_ref, w_ref, o_ref, bdotb_ref, iota_ref, *,
                        n_tokens):
    @pl.when(pl.program_id(0) == 0)
    def _init():
        w0 = w_ref[...]
        bdotb_ref[...] = jnp.sum(w0 * w0, axis=1)[None, :]
        iota_ref[...] = lax.broadcasted_iota(
            jnp.int32, iota_ref.shape, 1).astype(jnp.float32)

    x = x_ref[...]
    xs = -(x + x)
    mm = lax.dot_general(
        xs, w_ref[...], dimension_numbers=(((1,), (1,)), ((), ())),
        preferred_element_type=jnp.float32)
    adota = jnp.sum(x * x, axis=1, keepdims=True)
    dist = (adota + mm) + bdotb_ref[...]
    m = jnp.min(dist, axis=1, keepdims=True)
    idxf = jnp.min(
        jnp.where(dist == m, iota_ref[...], float(n_tokens)), axis=1)
    o_ref[0, 0, :] = idxf.astype(jnp.int32)


def kernel(X, weight):
    B, S, D = X.shape
    K = weight.shape[0]
    M = B * S
    x2 = X.reshape(M, D)

    BM = 2304
    nblk = M // BM

    out = pl.pallas_call(
        functools.partial(_nearest_idx_kernel, n_tokens=K),
        grid=(nblk,),
        in_specs=[
            pl.BlockSpec((BM, D), lambda i: (i, 0)),
            pl.BlockSpec((K, D), lambda i: (0, 0)),
        ],
        out_specs=pl.BlockSpec((1, 1, BM), lambda i: (i, 0, 0)),
        out_shape=jax.ShapeDtypeStruct((nblk, 1, BM), jnp.int32),
        scratch_shapes=[
            pltpu.VMEM((1, K), jnp.float32),
            pltpu.VMEM((BM, K), jnp.float32),
        ],
        compiler_params=pltpu.CompilerParams(
            dimension_semantics=("arbitrary",)),
    )(x2, weight)
    return out.reshape(B, S)


# speedup vs baseline: 1.0343x; 1.0343x over previous
import functools

import jax
import jax.numpy as jnp
from jax import lax
from jax.experimental import pallas as pl
from jax.experimental.pallas import tpu as pltpu


def _nearest_idx_kernel(x_ref, w_ref, o_ref, bdotb_ref, iota_ref, *,
                        n_tokens):
    @pl.when(pl.program_id(0) == 0)
    def _init():
        w0 = w_ref[...]
        bdotb_ref[...] = jnp.sum(w0 * w0, axis=1)[None, :]
        iota_ref[...] = lax.broadcasted_iota(
            jnp.int32, iota_ref.shape, 1).astype(jnp.float32)

    x = x_ref[...]
    xs = -(x + x)
    mm = lax.dot_general(
        xs, w_ref[...], dimension_numbers=(((1,), (1,)), ((), ())),
        preferred_element_type=jnp.float32)
    adota = jnp.sum(x * x, axis=1, keepdims=True)
    dist = (adota + mm) + bdotb_ref[...]
    o_ref[0, 0, :] = jnp.argmin(dist, axis=1).astype(jnp.int32)


def kernel(X, weight):
    B, S, D = X.shape
    K = weight.shape[0]
    M = B * S
    x2 = X.reshape(M, D)

    BM = 2304
    nblk = M // BM

    out = pl.pallas_call(
        functools.partial(_nearest_idx_kernel, n_tokens=K),
        grid=(nblk,),
        in_specs=[
            pl.BlockSpec((BM, D), lambda i: (i, 0)),
            pl.BlockSpec((K, D), lambda i: (0, 0)),
        ],
        out_specs=pl.BlockSpec((1, 1, BM), lambda i: (i, 0, 0)),
        out_shape=jax.ShapeDtypeStruct((nblk, 1, BM), jnp.int32),
        scratch_shapes=[
            pltpu.VMEM((1, K), jnp.float32),
            pltpu.VMEM((BM, K), jnp.float32),
        ],
        compiler_params=pltpu.CompilerParams(
            dimension_semantics=("arbitrary",)),
    )(x2, weight)
    return out.reshape(B, S)
